# baseline (speedup 1.0000x reference)
import functools

import jax
import jax.numpy as jnp
from jax import lax
from jax.experimental import pallas as pl
from jax.experimental.pallas import tpu as pltpu
from jax.experimental.pallas import tpu_sc as plsc

_E = 16
_B = 32
_DIN = 2048
_HW = 256
_D = 8192
_N = 64
_NC = 16
_DBLK = _D // _NC


def _phase1_kernel(x_ref, wrt_ref, br_ref, wnt_ref, bn_ref, eps_ref,
                   noisy_ref, z_ref, mh_ref):
    i = pl.program_id(0)

    x = x_ref[0]
    mh_ref[pl.ds(i, 1), :] = jnp.sum(x, axis=0)[None, :] * (1.0 / _HW)

    @pl.when(i == _B - 1)
    def _route():
        mh = mh_ref[...]
        dn = (((1,), (1,)), ((), ()))
        lin_r = lax.dot_general(mh, wrt_ref[...], dn,
                                preferred_element_type=jnp.float32) + br_ref[...]
        p = jax.nn.softmax(lin_r, axis=1)
        lin_n = lax.dot_general(mh, wnt_ref[...], dn,
                                preferred_element_type=jnp.float32) + bn_ref[...]
        s = eps_ref[...] * jax.nn.softplus(lin_n)
        q = jax.nn.softmax(s, axis=1)
        noisy = p + q
        noisy_ref[...] = noisy
        zl = jnp.log(jnp.sum(jnp.exp(noisy), axis=1, keepdims=True))
        z_ref[0, 0] = jnp.sum(zl * zl) * (1.0 / _B)


def _sc_route_body(noisy_hbm, bins_hbm, row_v, out_v):
    wid = lax.axis_index("s") * 2 + lax.axis_index("c")
    pltpu.sync_copy(noisy_hbm.at[wid], row_v)
    r = row_v[...]
    iota = lax.broadcasted_iota(jnp.int32, (_E,), 0)
    m1 = jnp.max(r)
    i1 = jnp.min(jnp.where(r == m1, iota, _E))
    rest = jnp.where(iota == i1, -1e30, r)
    m2 = jnp.max(rest)
    i2 = jnp.min(jnp.where(rest == m2, iota, _E))
    dv = jnp.full((_E,), m2 - m1, jnp.float32)
    t = jnp.exp(dv)
    w1 = 1.0 / (1.0 + t)
    w2 = 1.0 - w1
    out_v[...] = (jnp.where(iota == i1, w1, 0.0)
                  + jnp.where(iota == i2, w2, 0.0))
    pltpu.sync_copy(out_v, bins_hbm.at[wid])


def _combine_kernel(bins_ref, at_ref, combt_ref, w_ref):
    i = pl.program_id(0)

    @pl.when(i == 0)
    def _weights():
        binsum = jnp.sum(bins_ref[...], axis=0, keepdims=True) * (1.0 / _B)
        iota_r = lax.broadcasted_iota(jnp.int32, (1, _E), 1)
        for e in range(_E):
            w_ref[0, e] = jnp.sum(jnp.where(iota_r == e, binsum, 0.0))

    acc = w_ref[0, 0] * at_ref[0]
    for e in range(1, _E):
        acc = acc + w_ref[0, e] * at_ref[e]
    combt_ref[...] = acc


@functools.partial(jax.jit, static_argnums=())
def kernel(inputs, W_route, b_route, W_noise, b_noise, A_logs, noise_eps):
    x4 = jnp.transpose(inputs, (0, 2, 3, 1)).reshape(_B, _HW, _DIN)
    wrt = W_route.T
    wnt = W_noise.T
    at = jnp.transpose(A_logs, (0, 2, 1))
    br = b_route.reshape(1, _E)
    bn = b_noise.reshape(1, _E)

    noisy, z = pl.pallas_call(
        _phase1_kernel,
        grid=(_B,),
        in_specs=[
            pl.BlockSpec((1, _HW, _DIN), lambda i: (i, 0, 0)),
            pl.BlockSpec((_E, _DIN), lambda i: (0, 0)),
            pl.BlockSpec((1, _E), lambda i: (0, 0)),
            pl.BlockSpec((_E, _DIN), lambda i: (0, 0)),
            pl.BlockSpec((1, _E), lambda i: (0, 0)),
            pl.BlockSpec((_B, _E), lambda i: (0, 0)),
        ],
        out_specs=[
            pl.BlockSpec((_B, _E), lambda i: (0, 0)),
            pl.BlockSpec(memory_space=pltpu.SMEM),
        ],
        out_shape=[
            jax.ShapeDtypeStruct((_B, _E), jnp.float32),
            jax.ShapeDtypeStruct((1, 1), jnp.float32),
        ],
        scratch_shapes=[
            pltpu.VMEM((_B, _DIN), jnp.float32),
        ],
    )(x4, wrt, br, wnt, bn, noise_eps)

    sc_route = functools.partial(
        pl.kernel,
        out_type=jax.ShapeDtypeStruct((_B, _E), jnp.float32),
        mesh=plsc.VectorSubcoreMesh(core_axis_name="c", subcore_axis_name="s"),
        scratch_types=[
            pltpu.VMEM((_E,), jnp.float32),
            pltpu.VMEM((_E,), jnp.float32),
        ],
        compiler_params=pltpu.CompilerParams(needs_layout_passes=False),
    )(_sc_route_body)
    bins = sc_route(noisy)

    combt = pl.pallas_call(
        _combine_kernel,
        grid=(_NC,),
        in_specs=[
            pl.BlockSpec((_B, _E), lambda i: (0, 0)),
            pl.BlockSpec((_E, _N, _DBLK), lambda i: (0, 0, i)),
        ],
        out_specs=pl.BlockSpec((_N, _DBLK), lambda i: (0, i)),
        out_shape=jax.ShapeDtypeStruct((_N, _D), jnp.float32),
        scratch_shapes=[
            pltpu.SMEM((1, _E), jnp.float32),
        ],
    )(bins, at)

    combined = combt.T
    z_loss = z.reshape(())
    return (combined, z_loss)
